# baseline (speedup 1.0000x reference)
import functools

import jax
import jax.numpy as jnp
from jax import lax
from jax.experimental import pallas as pl
from jax.experimental.pallas import tpu as pltpu
from jax.experimental.pallas import tpu_sc as plsc

E = 320000
D = 128
V0, V1, V2 = 12, 27, 7
C = V0 * V1 * V2

NC, NS = 2, 16
NW = NC * NS
PER_W = E // NW
CHUNK = 80
STEPS = PER_W // CHUNK
LAST = STEPS - 1
NBUF = 10
LEAD = 5


def _ftab_body(t0_ref, t1_ref, t2_ref, ftab_ref):
    r = lax.broadcasted_iota(jnp.int32, (C, 1), 0)
    a = r // (V1 * V2)
    b = (r // V2) % V1
    c = r % V2
    oh0 = (a == lax.broadcasted_iota(jnp.int32, (C, V0), 1)).astype(jnp.float32)
    oh1 = (b == lax.broadcasted_iota(jnp.int32, (C, V1), 1)).astype(jnp.float32)
    oh2 = (c == lax.broadcasted_iota(jnp.int32, (C, V2), 1)).astype(jnp.float32)
    hi = lax.Precision.HIGHEST
    ftab_ref[:] = (
        jnp.dot(oh0, t0_ref[:], preferred_element_type=jnp.float32, precision=hi)
        + jnp.dot(oh1, t1_ref[:], preferred_element_type=jnp.float32, precision=hi)
        + jnp.dot(oh2, t2_ref[:], preferred_element_type=jnp.float32, precision=hi)
    )


_ftab = pl.pallas_call(
    _ftab_body,
    out_shape=jax.ShapeDtypeStruct((C, D), jnp.float32),
)


def _sc_gather_fn():
    mesh = plsc.VectorSubcoreMesh(
        core_axis_name="c", subcore_axis_name="s",
        num_cores=NC, num_subcores=NS)

    @functools.partial(
        pl.kernel,
        mesh=mesh,
        out_type=jax.ShapeDtypeStruct((E, D), jnp.float32),
        scratch_types=[
            pltpu.VMEM((NBUF * CHUNK,), jnp.int32),
            pltpu.VMEM((NBUF * CHUNK,), jnp.int32),
            pltpu.VMEM((NBUF * CHUNK,), jnp.int32),
            pltpu.VMEM((NBUF * CHUNK,), jnp.int32),
            pltpu.VMEM((NBUF, CHUNK, D), jnp.float32),
            pltpu.VMEM_SHARED((C, D), jnp.float32),
            [pltpu.SemaphoreType.DMA] * NBUF,
            [pltpu.SemaphoreType.DMA] * NBUF,
            [pltpu.SemaphoreType.DMA] * NBUF,
        ],
    )
    def sc_gather(ftab_hbm, f0_hbm, f1_hbm, f2_hbm, out_hbm,
                  f0c, f1c, f2c, idxc, rows_v, ftab_spm,
                  gsem, ssem, fsem):
        sid = lax.axis_index("s")
        wid = sid * NC + lax.axis_index("c")
        base = wid * PER_W

        def issue_f(s, slot):
            off = base + s * CHUNK
            d = pl.ds(slot * CHUNK, CHUNK)
            for src, dst in ((f0_hbm, f0c), (f1_hbm, f1c), (f2_hbm, f2c)):
                pltpu.async_copy(src.at[pl.ds(off, CHUNK)], dst.at[d], fsem[slot])

        def drain_f(slot):
            d = pl.ds(slot * CHUNK, CHUNK)
            for dst in (f0c, f1c, f2c):
                pltpu.make_async_copy(
                    f0_hbm.at[pl.ds(0, CHUNK)], dst.at[d], fsem[slot]).wait()

        def compute_cidx(slot):
            for q in range(CHUNK // 16):
                d = pl.ds(slot * CHUNK + q * 16, 16)
                idxc[d] = f0c[d] * (V1 * V2) + f1c[d] * V2 + f2c[d]

        def issue_gather(slot):
            pltpu.async_copy(
                ftab_spm.at[idxc.at[pl.ds(slot * CHUNK, CHUNK)]],
                rows_v.at[slot], gsem[slot])

        def drain_gather(slot):
            pltpu.make_async_copy(
                out_hbm.at[pl.ds(0, CHUNK)], rows_v.at[slot], gsem[slot]).wait()

        def start_store(s, slot):
            pltpu.make_async_copy(
                rows_v.at[slot],
                out_hbm.at[pl.ds(base + s * CHUNK, CHUNK)],
                ssem[slot]).start()

        def drain_store(slot):
            pltpu.make_async_copy(
                out_hbm.at[pl.ds(0, CHUNK)], rows_v.at[slot], ssem[slot]).wait()

        def step(s, p, sdrain=True, gmore=True, fmore=True):
            q = (p + LEAD) % NBUF
            drain_gather(p)
            start_store(s, p)
            if gmore:
                drain_f(q)
                compute_cidx(q)
            if sdrain:
                drain_store(q)
            if gmore:
                issue_gather(q)
            if fmore:
                issue_f(s + NBUF, p)

        @pl.when(sid == 0)
        def _():
            pltpu.sync_copy(ftab_hbm, ftab_spm)
        for s in range(NBUF):
            issue_f(s, s)
        for s in range(LEAD):
            drain_f(s)
            compute_cidx(s)
        plsc.subcore_barrier()
        for s in range(LEAD):
            issue_gather(s)

        for s in range(NBUF - LEAD):
            step(s, s, sdrain=False)

        def body(i, carry):
            s = NBUF * i + (NBUF - LEAD)
            for j in range(NBUF):
                step(s + j, (NBUF - LEAD + j) % NBUF)
            return carry

        lax.fori_loop(0, (STEPS - NBUF - (NBUF - LEAD)) // NBUF, body, 0)

        for s in range(STEPS - NBUF, STEPS):
            step(s, s % NBUF,
                 gmore=(s + LEAD <= LAST),
                 fmore=(s + NBUF <= LAST))
        for s in range(STEPS - (NBUF - LEAD), STEPS):
            drain_store(s % NBUF)

    return sc_gather


_sc_gather = _sc_gather_fn()


def kernel(edge_feat_0, edge_feat_1, edge_feat_2, table_0, table_1, table_2):
    ftab = _ftab(table_0, table_1, table_2)
    return _sc_gather(ftab, edge_feat_0, edge_feat_1, edge_feat_2)


# speedup vs baseline: 1.0056x; 1.0056x over previous
import functools

import jax
import jax.numpy as jnp
from jax import lax
from jax.experimental import pallas as pl
from jax.experimental.pallas import tpu as pltpu
from jax.experimental.pallas import tpu_sc as plsc

E = 320000
D = 128
V0, V1, V2 = 12, 27, 7
C = V0 * V1 * V2

NC, NS = 2, 16
NW = NC * NS
PER_W = E // NW
CHUNK = 80
STEPS = PER_W // CHUNK
LAST = STEPS - 1
NBUF = 8
LEAD = 3


def _ftab_body(t0_ref, t1_ref, t2_ref, ftab_ref):
    r = lax.broadcasted_iota(jnp.int32, (C, 1), 0)
    a = r // (V1 * V2)
    b = (r // V2) % V1
    c = r % V2
    oh0 = (a == lax.broadcasted_iota(jnp.int32, (C, V0), 1)).astype(jnp.float32)
    oh1 = (b == lax.broadcasted_iota(jnp.int32, (C, V1), 1)).astype(jnp.float32)
    oh2 = (c == lax.broadcasted_iota(jnp.int32, (C, V2), 1)).astype(jnp.float32)
    hi = lax.Precision.HIGHEST
    ftab_ref[:] = (
        jnp.dot(oh0, t0_ref[:], preferred_element_type=jnp.float32, precision=hi)
        + jnp.dot(oh1, t1_ref[:], preferred_element_type=jnp.float32, precision=hi)
        + jnp.dot(oh2, t2_ref[:], preferred_element_type=jnp.float32, precision=hi)
    )


_ftab = pl.pallas_call(
    _ftab_body,
    out_shape=jax.ShapeDtypeStruct((C, D), jnp.float32),
)


def _sc_gather_fn():
    mesh = plsc.VectorSubcoreMesh(
        core_axis_name="c", subcore_axis_name="s",
        num_cores=NC, num_subcores=NS)

    @functools.partial(
        pl.kernel,
        mesh=mesh,
        out_type=jax.ShapeDtypeStruct((E, D), jnp.float32),
        scratch_types=[
            pltpu.VMEM((NBUF * CHUNK,), jnp.int32),
            pltpu.VMEM((NBUF * CHUNK,), jnp.int32),
            pltpu.VMEM((NBUF * CHUNK,), jnp.int32),
            pltpu.VMEM((NBUF * CHUNK,), jnp.int32),
            pltpu.VMEM((NBUF, CHUNK, D), jnp.float32),
            pltpu.VMEM_SHARED((C, D), jnp.float32),
            [pltpu.SemaphoreType.DMA] * NBUF,
            [pltpu.SemaphoreType.DMA] * NBUF,
            [pltpu.SemaphoreType.DMA] * NBUF,
        ],
    )
    def sc_gather(ftab_hbm, f0_hbm, f1_hbm, f2_hbm, out_hbm,
                  f0c, f1c, f2c, idxc, rows_v, ftab_spm,
                  gsem, ssem, fsem):
        sid = lax.axis_index("s")
        wid = sid * NC + lax.axis_index("c")
        base = wid * PER_W

        def issue_f(s, slot):
            off = base + s * CHUNK
            d = pl.ds(slot * CHUNK, CHUNK)
            for src, dst in ((f0_hbm, f0c), (f1_hbm, f1c), (f2_hbm, f2c)):
                pltpu.async_copy(src.at[pl.ds(off, CHUNK)], dst.at[d], fsem[slot])

        def drain_f(slot):
            d = pl.ds(slot * CHUNK, CHUNK)
            for dst in (f0c, f1c, f2c):
                pltpu.make_async_copy(
                    f0_hbm.at[pl.ds(0, CHUNK)], dst.at[d], fsem[slot]).wait()

        def compute_cidx(slot):
            for q in range(CHUNK // 16):
                d = pl.ds(slot * CHUNK + q * 16, 16)
                idxc[d] = f0c[d] * (V1 * V2) + f1c[d] * V2 + f2c[d]

        def issue_gather(slot):
            pltpu.async_copy(
                ftab_spm.at[idxc.at[pl.ds(slot * CHUNK, CHUNK)]],
                rows_v.at[slot], gsem[slot])

        def drain_gather(slot):
            pltpu.make_async_copy(
                out_hbm.at[pl.ds(0, CHUNK)], rows_v.at[slot], gsem[slot]).wait()

        def start_store(s, slot):
            pltpu.make_async_copy(
                rows_v.at[slot],
                out_hbm.at[pl.ds(base + s * CHUNK, CHUNK)],
                ssem[slot]).start()

        def drain_store(slot):
            pltpu.make_async_copy(
                out_hbm.at[pl.ds(0, CHUNK)], rows_v.at[slot], ssem[slot]).wait()

        def step(s, p, sdrain=True, gmore=True, fmore=True):
            q = (p + LEAD) % NBUF
            drain_gather(p)
            start_store(s, p)
            if gmore:
                drain_f(q)
                compute_cidx(q)
            if sdrain:
                drain_store(q)
            if gmore:
                issue_gather(q)
            if fmore:
                issue_f(s + NBUF, p)

        @pl.when(sid == 0)
        def _():
            pltpu.sync_copy(ftab_hbm, ftab_spm)
        for s in range(NBUF):
            issue_f(s, s)
        for s in range(LEAD):
            drain_f(s)
            compute_cidx(s)
        plsc.subcore_barrier()
        for s in range(LEAD):
            issue_gather(s)

        for s in range(NBUF - LEAD):
            step(s, s, sdrain=False)

        def body(i, carry):
            s = NBUF * i + (NBUF - LEAD)
            for j in range(NBUF):
                step(s + j, (NBUF - LEAD + j) % NBUF)
            return carry

        lax.fori_loop(0, (STEPS - NBUF - (NBUF - LEAD)) // NBUF, body, 0)

        for s in range(STEPS - NBUF, STEPS):
            step(s, s % NBUF,
                 gmore=(s + LEAD <= LAST),
                 fmore=(s + NBUF <= LAST))
        for s in range(STEPS - (NBUF - LEAD), STEPS):
            drain_store(s % NBUF)

    return sc_gather


_sc_gather = _sc_gather_fn()


def kernel(edge_feat_0, edge_feat_1, edge_feat_2, table_0, table_1, table_2):
    ftab = _ftab(table_0, table_1, table_2)
    return _sc_gather(ftab, edge_feat_0, edge_feat_1, edge_feat_2)
